# baseline (speedup 1.0000x reference)
import jax
import jax.numpy as jnp
from jax.experimental import pallas as pl
from jax.experimental.pallas import tpu as pltpu

_B = 16384
_C = 1000
_BLK = 2048
_NBLK = _B // _BLK


def _loss_body(lab_ref, pred_ref, out_ref, acc_ref):
    i = pl.program_id(0)

    @pl.when(i == 0)
    def _():
        acc_ref[0, 0] = 0.0

    x = pred_ref[...]
    labT = lab_ref[0].reshape(_BLK, 1)
    cols = jax.lax.broadcasted_iota(jnp.int32, (_BLK, _C), 1)
    masked = jnp.where(cols == labT, x, 0.0)
    ones = jnp.ones((_C, 8), jnp.float32)
    sel = jax.lax.dot_general(masked, ones, (((1,), (0,)), ((), ())),
                              preferred_element_type=jnp.float32)
    acc_ref[0, 0] += jnp.sum(jnp.log(sel + 1e-8)) * 0.125

    @pl.when(i == _NBLK - 1)
    def _():
        out_ref[0, 0] = acc_ref[0, 0] * (-1.0 / (_B * _C))


def kernel(pred, label):
    lab3 = label.astype(jnp.int32).reshape(_NBLK, 1, _BLK)
    out = pl.pallas_call(
        _loss_body,
        grid=(_NBLK,),
        in_specs=[
            pl.BlockSpec((1, 1, _BLK), lambda i: (i, 0, 0)),
            pl.BlockSpec((_BLK, _C), lambda i: (i, 0)),
        ],
        out_specs=pl.BlockSpec(memory_space=pltpu.SMEM),
        out_shape=jax.ShapeDtypeStruct((1, 1), jnp.float32),
        scratch_shapes=[pltpu.SMEM((1, 1), jnp.float32)],
    )(lab3, pred)
    return out[0, 0]


# speedup vs baseline: 1.0222x; 1.0222x over previous
import jax
import jax.numpy as jnp
from jax.experimental import pallas as pl
from jax.experimental.pallas import tpu as pltpu

_B = 16384
_C = 1000
_BLK = 1024
_NSPLIT = 4
_NSTEP = _B // (_BLK * _NSPLIT)


def _loss_body(lab_ref, *refs):
    pred_refs = refs[:_NSPLIT]
    out_ref, acc_ref = refs[_NSPLIT], refs[_NSPLIT + 1]
    i = pl.program_id(0)

    @pl.when(i == 0)
    def _():
        acc_ref[0, 0] = 0.0

    cols = jax.lax.broadcasted_iota(jnp.int32, (_BLK, _C), 1)
    ones = jnp.ones((_C, 8), jnp.float32)
    part = 0.0
    for q in range(_NSPLIT):
        x = pred_refs[q][...]
        labT = lab_ref[0, q].reshape(_BLK, 1)
        masked = jnp.where(cols == labT, x, 0.0)
        sel = jax.lax.dot_general(masked, ones, (((1,), (0,)), ((), ())),
                                  preferred_element_type=jnp.float32)
        part += jnp.sum(jnp.log(sel + 1e-8)) * 0.125
    acc_ref[0, 0] += part

    @pl.when(i == _NSTEP - 1)
    def _():
        out_ref[0, 0] = acc_ref[0, 0] * (-1.0 / (_B * _C))


def kernel(pred, label):
    lab3 = label.astype(jnp.int32).reshape(
        _NSPLIT, _NSTEP, _BLK).transpose(1, 0, 2)
    qrows = _B // _NSPLIT // _BLK
    in_specs = [pl.BlockSpec((1, _NSPLIT, _BLK), lambda i: (i, 0, 0))]
    for q in range(_NSPLIT):
        in_specs.append(
            pl.BlockSpec((_BLK, _C), lambda i, q=q: (q * qrows + i, 0)))
    out = pl.pallas_call(
        _loss_body,
        grid=(_NSTEP,),
        in_specs=in_specs,
        out_specs=pl.BlockSpec(memory_space=pltpu.SMEM),
        out_shape=jax.ShapeDtypeStruct((1, 1), jnp.float32),
        scratch_shapes=[pltpu.SMEM((1, 1), jnp.float32)],
    )(lab3, pred, pred, pred, pred)
    return out[0, 0]
